# baseline (speedup 1.0000x reference)
import functools

import jax
import jax.numpy as jnp
import numpy as np
from jax import lax
from jax.experimental import pallas as pl
from jax.experimental.pallas import tpu as pltpu
from jax.experimental.pallas import tpu_sc as plsc

N_NODE = 50000
SP_NN = 50000
E_PL = 800000
E_NX = 100000
HID = 64
FH = HID // 2
NSUB = 16

_CHUNK = {E_PL: (128, 10), E_NX: (80, 10)}

_ZS = 625


def _norm_np():
    return {
        'u': np.array([[389.42752, 172.90794, 147.81108, 4.5563765], [147.1627, 78.01324, 228.31424, 2.2156637]], dtype=np.float32),
        'v': np.array([[368.83023, 173.01247, 154.14513, 4.449338], [145.29645, 80.54078, 282.34027, 1.8969047]], dtype=np.float32),
        'y': np.array([[546.2973, 172.77615, 116.974, 4.1647816], [283.47656, 73.99135, 115.49256, 1.4615369]], dtype=np.float32),
    }


_NORM = _norm_np()


def _zero_zbuf(zbuf):
    def row(i, carry):
        for j in range(FH // 16):
            zbuf[i, pl.ds(j * 16, 16)] = jnp.zeros((16,), jnp.float32)
        return carry
    lax.fori_loop(0, _ZS, row, 0)


def _zero_acc(acc, zbuf, s):
    for t in range(N_NODE // _ZS // NSUB):
        sl = s * (N_NODE // _ZS // NSUB) + t
        pltpu.sync_copy(zbuf, acc.at[pl.ds(sl * _ZS, _ZS)])


def _accum(table, src2, dst2, src_vs, dst_vs, rows, acc, sems, s, E):
    K, S = _CHUNK[E]
    gsems, ssems = sems[:2], sems[2:]
    nsuper = E // (K * S)
    cnt = (nsuper - s + NSUB - 1) // NSUB

    def body(k, carry):
        sg = s + k * NSUB
        pltpu.sync_copy(src2.at[pl.ds(sg * S, S)], src_vs)
        pltpu.sync_copy(dst2.at[pl.ds(sg * S, S)], dst_vs)
        g = [None, None]
        g[0] = pltpu.async_copy(table.at[src_vs.at[0]], rows[0], gsems[0])
        for j in range(S):
            b = j & 1
            if j + 1 < S:
                g[b ^ 1] = pltpu.async_copy(
                    table.at[src_vs.at[j + 1]], rows[b ^ 1], gsems[b ^ 1])
            g[b].wait()
            pltpu.sync_copy(rows[b], acc.at[dst_vs.at[j]], add=True)
        return carry
    lax.fori_loop(0, cnt, body, 0)


def _writeout(acc, out_hbm, s):
    n = N_NODE // NSUB
    pltpu.sync_copy(acc.at[pl.ds(s * n, n)], out_hbm.at[pl.ds(s * n, n)])


def _phase(tables, src2, dst2, out, scr, c, s, E, zero_first):
    src_vs, dst_vs, r0, r1, zbuf, acc, sems = scr
    if zero_first:
        _zero_acc(acc, zbuf, s)
        plsc.subcore_barrier()

    @pl.when(c == 0)
    def _():
        _accum(tables[0], src2, dst2, src_vs, dst_vs, (r0, r1), acc, sems, s, E)

    @pl.when(c == 1)
    def _():
        _accum(tables[1], src2, dst2, src_vs, dst_vs, (r0, r1), acc, sems, s, E)

    if out is not None:
        plsc.subcore_barrier()

        @pl.when(c == 0)
        def _():
            _writeout(acc, out[0], s)

        @pl.when(c == 1)
        def _():
            _writeout(acc, out[1], s)

        plsc.subcore_barrier()


def _seg_plane_body(E):
    def body(t0l, t0h, t1l, t1h, t2l, t2h, s0, s1, s2, d0, d1, d2,
             o0l, o0h, o1l, o1h, o2l, o2h,
             src_vs, dst_vs, r0, r1, zbuf, acc, m0, m1, m2, m3):
        c = lax.axis_index("c")
        s = lax.axis_index("s")
        scr = (src_vs, dst_vs, r0, r1, zbuf, acc, (m0, m1, m2, m3))
        _zero_zbuf(zbuf)
        for tl, th, sr, dr, ol, oh in ((t0l, t0h, s0, d0, o0l, o0h),
                                       (t1l, t1h, s1, d1, o1l, o1h),
                                       (t2l, t2h, s2, d2, o2l, o2h)):
            _phase((tl, th), sr, dr, (ol, oh), scr, c, s, E, True)
    return body


def _seg_back_body(E):
    def body(t_lo, t_hi, s0, s1, s2, d0, d1, d2,
             o0l, o0h, o1l, o1h, o2l, o2h,
             src_vs, dst_vs, r0, r1, zbuf, acc, m0, m1, m2, m3):
        c = lax.axis_index("c")
        s = lax.axis_index("s")
        scr = (src_vs, dst_vs, r0, r1, zbuf, acc, (m0, m1, m2, m3))
        _zero_zbuf(zbuf)
        for sr, dr, ol, oh in ((s0, d0, o0l, o0h), (s1, d1, o1l, o1h),
                               (s2, d2, o2l, o2h)):
            _phase((t_lo, t_hi), sr, dr, (ol, oh), scr, c, s, E, True)
    return body


def _seg3_body(E):
    def body(tu_lo, tu_hi, tv_lo, tv_hi, ty_lo, ty_hi,
             s0, s1, s2, d0, d1, d2, out_lo, out_hi,
             src_vs, dst_vs, r0, r1, zbuf, acc, m0, m1, m2, m3):
        c = lax.axis_index("c")
        s = lax.axis_index("s")
        scr = (src_vs, dst_vs, r0, r1, zbuf, acc, (m0, m1, m2, m3))
        _zero_zbuf(zbuf)
        _zero_acc(acc, zbuf, s)
        plsc.subcore_barrier()
        _phase((tu_lo, tu_hi), s0, d0, None, scr, c, s, E, False)
        _phase((tv_lo, tv_hi), s1, d1, None, scr, c, s, E, False)
        _phase((ty_lo, ty_hi), s2, d2, None, scr, c, s, E, False)
        plsc.subcore_barrier()

        @pl.when(c == 0)
        def _():
            _writeout(acc, out_lo, s)

        @pl.when(c == 1)
        def _():
            _writeout(acc, out_hi, s)
    return body


def _sc_scratch(E):
    K, S = _CHUNK[E]
    return [
        pltpu.VMEM((S, K), jnp.int32),
        pltpu.VMEM((S, K), jnp.int32),
        pltpu.VMEM((K, FH), jnp.float32),
        pltpu.VMEM((K, FH), jnp.float32),
        pltpu.VMEM((_ZS, FH), jnp.float32),
        pltpu.VMEM_SHARED((N_NODE, FH), jnp.float32),
        pltpu.SemaphoreType.DMA,
        pltpu.SemaphoreType.DMA,
        pltpu.SemaphoreType.DMA,
        pltpu.SemaphoreType.DMA,
    ]


def _mesh():
    return plsc.VectorSubcoreMesh(core_axis_name="c", subcore_axis_name="s",
                                  num_cores=2, num_subcores=NSUB)


_SC_PARAMS = pltpu.CompilerParams(use_tc_tiling_on_sc=False)

_HALF = jax.ShapeDtypeStruct((N_NODE, FH), jnp.float32)


@functools.cache
def _seg_plane(E):
    return pl.kernel(
        _seg_plane_body(E),
        out_type=(_HALF,) * 6,
        mesh=_mesh(),
        scratch_types=_sc_scratch(E),
        compiler_params=_SC_PARAMS,
    )


@functools.cache
def _seg_back(E):
    return pl.kernel(
        _seg_back_body(E),
        out_type=(_HALF,) * 6,
        mesh=_mesh(),
        scratch_types=_sc_scratch(E),
        compiler_params=_SC_PARAMS,
    )


@functools.cache
def _seg3(E):
    return pl.kernel(
        _seg3_body(E),
        out_type=(_HALF, _HALF),
        mesh=_mesh(),
        scratch_types=_sc_scratch(E),
        compiler_params=_SC_PARAMS,
    )


_BR = 2000
_GRID = N_NODE // _BR


def _rows(d):
    return pl.BlockSpec((_BR, d), lambda i: (i, 0))


def _whole(shape):
    return pl.BlockSpec(shape, lambda i: tuple(0 for _ in shape))


def _split_store(o, lo_ref, hi_ref):
    lo_ref[...] = o[:, :FH]
    hi_ref[...] = o[:, FH:]


def _enc_body(x_ref, p_ref, mu_ref, sd_ref, we_ref, b_ref, wm_ref,
              lo_ref, hi_ref, mlo_ref, mhi_ref):
    xn = (x_ref[...] - mu_ref[...]) / sd_ref[...]
    f = jnp.concatenate([xn, p_ref[...]], axis=1)
    h = jnp.tanh(jnp.dot(f, we_ref[...], preferred_element_type=jnp.float32)
                 + b_ref[...])
    _split_store(h, lo_ref, hi_ref)
    _split_store(jnp.dot(h, wm_ref[...], preferred_element_type=jnp.float32),
                 mlo_ref, mhi_ref)


@jax.jit
def _enc(x, pos, mu, sd, we, b, wm):
    return pl.pallas_call(
        _enc_body,
        grid=(_GRID,),
        in_specs=[_rows(4), _rows(2), _whole((1, 4)), _whole((1, 4)),
                  _whole((6, HID)), _whole((1, HID)), _whole((HID, HID))],
        out_specs=[_rows(FH)] * 4,
        out_shape=[_HALF] * 4,
    )(x, pos, mu, sd, we, b, wm)


def _upd_body(hl_ref, hh_ref, al_ref, ah_ref, ws_ref, wn_ref,
              lo_ref, hi_ref, nlo_ref, nhi_ref):
    h = jnp.concatenate([hl_ref[...], hh_ref[...]], axis=1)
    a = jnp.concatenate([al_ref[...], ah_ref[...]], axis=1)
    nh = jnp.tanh(jnp.dot(h, ws_ref[...], preferred_element_type=jnp.float32) + a)
    _split_store(nh, lo_ref, hi_ref)
    _split_store(jnp.dot(nh, wn_ref[...], preferred_element_type=jnp.float32),
                 nlo_ref, nhi_ref)


@jax.jit
def _upd(hl, hh, al, ah, ws, wn):
    return pl.pallas_call(
        _upd_body,
        grid=(_GRID,),
        in_specs=[_rows(FH)] * 4 + [_whole((HID, HID))] * 2,
        out_specs=[_rows(FH)] * 4,
        out_shape=[_HALF] * 4,
    )(hl, hh, al, ah, ws, wn)


def _spupd_body(spz_ref, pl_ref, ph_ref, wb_ref,
                lo_ref, hi_ref, blo_ref, bhi_ref):
    p = jnp.concatenate([pl_ref[...], ph_ref[...]], axis=1)
    sp = jnp.tanh(p + spz_ref[0])
    _split_store(sp, lo_ref, hi_ref)
    _split_store(jnp.dot(sp, wb_ref[...], preferred_element_type=jnp.float32),
                 blo_ref, bhi_ref)


@jax.jit
def _spupd(spz, prel, preh, wb):
    return pl.pallas_call(
        _spupd_body,
        grid=(_GRID,),
        in_specs=[pl.BlockSpec(memory_space=pltpu.SMEM),
                  _rows(FH), _rows(FH), _whole((HID, HID))],
        out_specs=[_rows(FH)] * 4,
        out_shape=[_HALF] * 4,
    )(spz, prel, preh, wb)


def _back_body(hl_ref, hh_ref, bl_ref, bh_ref, wm_ref,
               lo_ref, hi_ref, mlo_ref, mhi_ref):
    h = jnp.concatenate([hl_ref[...], hh_ref[...]], axis=1)
    b = jnp.concatenate([bl_ref[...], bh_ref[...]], axis=1)
    nh = jnp.tanh(h + b)
    _split_store(nh, lo_ref, hi_ref)
    _split_store(jnp.dot(nh, wm_ref[...], preferred_element_type=jnp.float32),
                 mlo_ref, mhi_ref)


@jax.jit
def _backupd(hl, hh, bl, bh, wm):
    return pl.pallas_call(
        _back_body,
        grid=(_GRID,),
        in_specs=[_rows(FH)] * 4 + [_whole((HID, HID))],
        out_specs=[_rows(FH)] * 4,
        out_shape=[_HALF] * 4,
    )(hl, hh, bl, bh, wm)


def _heads_body(hl_ref, hh_ref, w6_ref, o_ref):
    h = jnp.concatenate([hl_ref[...], hh_ref[...]], axis=1)
    o_ref[...] = jnp.dot(h, w6_ref[...], preferred_element_type=jnp.float32)


@jax.jit
def _heads(hl, hh, w6):
    return pl.pallas_call(
        _heads_body,
        grid=(_GRID,),
        in_specs=[_rows(FH), _rows(FH), _whole((HID, 6))],
        out_specs=_rows(6),
        out_shape=jax.ShapeDtypeStruct((N_NODE, 6), jnp.float32),
    )(hl, hh, w6)


def _evt_body(sl_ref, sh_ref, we_ref, e_ref, v_ref):
    i = pl.program_id(0)

    @pl.when(i == 0)
    def _():
        e_ref[...] = jnp.zeros_like(e_ref)

    sp = jnp.concatenate([sl_ref[...], sh_ref[...]], axis=1)
    e_ref[...] += jnp.sum(sp, axis=0, keepdims=True)

    @pl.when(i == _GRID - 1)
    def _():
        e = e_ref[...] / np.float32(SP_NN)
        e_ref[...] = e
        v_ref[...] = jnp.dot(e, we_ref[...], preferred_element_type=jnp.float32)


@jax.jit
def _evt(sl, sh, we):
    return pl.pallas_call(
        _evt_body,
        grid=(_GRID,),
        in_specs=[_rows(FH), _rows(FH), _whole((HID, 5))],
        out_specs=[_whole((1, HID)), _whole((1, 5))],
        out_shape=[jax.ShapeDtypeStruct((1, HID), jnp.float32),
                   jax.ShapeDtypeStruct((1, 5), jnp.float32)],
    )(sl, sh, we)


def kernel(sp_num_nodes, u_x_dict, u_pos, v_x_dict, v_pos, y_x_dict, y_pos, evt_y,
           u_plane_u, u_nexus_sp, v_plane_v, v_nexus_sp, y_plane_y, y_nexus_sp,
           W_enc, b_enc, W_msg, W_self, W_nex, W_back, W_sem, W_filt, W_evt):
    f32, i32 = jnp.float32, jnp.int32
    planes = ('u', 'v', 'y')
    xs = {'u': u_x_dict, 'v': v_x_dict, 'y': y_x_dict}
    poss = {'u': u_pos, 'v': v_pos, 'y': y_pos}
    pe = {'u': u_plane_u.astype(i32), 'v': v_plane_v.astype(i32), 'y': y_plane_y.astype(i32)}
    ne = {'u': u_nexus_sp.astype(i32), 'v': v_nexus_sp.astype(i32), 'y': y_nexus_sp.astype(i32)}

    spz = (sp_num_nodes[0] - SP_NN).astype(f32).reshape(1)
    w2 = W_enc[4:6]
    b2 = b_enc.reshape(1, HID)

    mu = {p: jnp.asarray(_NORM[p][0]).reshape(1, 4) for p in planes}
    sd = {p: jnp.asarray(_NORM[p][1]).reshape(1, 4) for p in planes}
    hm = {}
    for p in planes:
        hm[p] = _enc(xs[p], poss[p], mu[p], sd[p], W_enc, b2, W_msg)

    Kp, Sp = _CHUNK[E_PL]
    Kn, Sn = _CHUNK[E_NX]
    ps = {p: pe[p][0].reshape(E_PL // Kp, Kp) for p in planes}
    pd = {p: pe[p][1].reshape(E_PL // Kp, Kp) for p in planes}
    ns = {p: ne[p][0].reshape(E_NX // Kn, Kn) for p in planes}
    nd = {p: ne[p][1].reshape(E_NX // Kn, Kn) for p in planes}

    for _ in range(3):
        ag = _seg_plane(E_PL)(hm['u'][2], hm['u'][3], hm['v'][2], hm['v'][3],
                              hm['y'][2], hm['y'][3],
                              ps['u'], ps['v'], ps['y'],
                              pd['u'], pd['v'], pd['y'])
        agg = {'u': ag[0:2], 'v': ag[2:4], 'y': ag[4:6]}
        hn = {p: _upd(hm[p][0], hm[p][1], agg[p][0], agg[p][1], W_self, W_nex)
              for p in planes}
        pre = _seg3(E_NX)(hn['u'][2], hn['u'][3], hn['v'][2], hn['v'][3],
                          hn['y'][2], hn['y'][3],
                          ns['u'], ns['v'], ns['y'],
                          nd['u'], nd['v'], nd['y'])
        sp = _spupd(spz, pre[0], pre[1], W_back)
        bk = _seg_back(E_NX)(sp[2], sp[3],
                             nd['u'], nd['v'], nd['y'],
                             ns['u'], ns['v'], ns['y'])
        back = {'u': bk[0:2], 'v': bk[2:4], 'y': bk[4:6]}
        hm = {p: _backupd(hn[p][0], hn[p][1], back[p][0], back[p][1], W_msg)
              for p in planes}

    w6 = jnp.concatenate([W_sem, W_filt], axis=1)
    x6 = {p: _heads(hm[p][0], hm[p][1], w6) for p in planes}
    e_evt, v_evt = _evt(sp[0], sp[1], W_evt)
    return (e_evt,
            x6['u'][:, :5], x6['v'][:, :5], x6['y'][:, :5],
            x6['u'][:, 5], x6['v'][:, 5], x6['y'][:, 5],
            v_evt)


# speedup vs baseline: 1.3534x; 1.3534x over previous
import functools

import jax
import jax.numpy as jnp
import numpy as np
from jax import lax
from jax.experimental import pallas as pl
from jax.experimental.pallas import tpu as pltpu
from jax.experimental.pallas import tpu_sc as plsc

N_NODE = 50000
SP_NN = 50000
E_PL = 800000
E_NX = 100000
HID = 64
FH = HID // 2
NSUB = 16

_CHUNK = {E_PL: (80, 5), E_NX: (80, 10)}

_ZS = 625


def _norm_np():
    return {
        'u': np.array([[389.42752, 172.90794, 147.81108, 4.5563765], [147.1627, 78.01324, 228.31424, 2.2156637]], dtype=np.float32),
        'v': np.array([[368.83023, 173.01247, 154.14513, 4.449338], [145.29645, 80.54078, 282.34027, 1.8969047]], dtype=np.float32),
        'y': np.array([[546.2973, 172.77615, 116.974, 4.1647816], [283.47656, 73.99135, 115.49256, 1.4615369]], dtype=np.float32),
    }


_NORM = _norm_np()


def _zero_buf(buf, n):
    def row(i, carry):
        for j in range(FH // 16):
            buf[i, pl.ds(j * 16, 16)] = jnp.zeros((16,), jnp.float32)
        return carry
    lax.fori_loop(0, n, row, 0)


def _zero_acc(acc, zbuf, rows0, s, E):
    n = N_NODE // NSUB
    base = s * n
    if _is_static(E):
        K, S = _CHUNK[E]
        W = K * S
        _zero_buf(rows0, W)
        src, w = rows0, W
    else:
        _zero_buf(zbuf, _ZS)
        src, w = zbuf, _ZS
    full, rem = n // w, n % w
    for t in range(full):
        pltpu.sync_copy(src, acc.at[pl.ds(base + t * w, w)])
    if rem:
        pltpu.sync_copy(src.at[pl.ds(0, rem)],
                        acc.at[pl.ds(base + full * w, rem)])


def _accum(table, src2, dst2, sv, dv, rows, acc, sems, s, E):
    K, S = _CHUNK[E]
    nsuper = E // (K * S)
    cnt = (nsuper - s + NSUB - 1) // NSUB

    def body(k, carry):
        sg = s + k * NSUB
        pltpu.sync_copy(src2.at[pl.ds(sg * S, S)], sv[0])
        pltpu.sync_copy(dst2.at[pl.ds(sg * S, S)], dv[0])
        g = [None, None]
        g[0] = pltpu.async_copy(table.at[sv[0].at[0]], rows[0].at[0], sems[0])
        for j in range(S):
            b = j & 1
            if j + 1 < S:
                g[b ^ 1] = pltpu.async_copy(
                    table.at[sv[0].at[j + 1]], rows[b ^ 1].at[0], sems[b ^ 1])
            g[b].wait()
            pltpu.sync_copy(rows[b].at[0], acc.at[dv[0].at[j]], add=True)
        return carry
    lax.fori_loop(0, cnt, body, 0)


def _accum_static(table, src1, dst1, sv, dv, rows, acc, sems, s, E):
    K, S = _CHUNK[E]
    W = K * S
    sup = E // (W * NSUB)

    def load(t, b):
        base = pl.multiple_of((s + t * NSUB) * W, 8)
        pltpu.sync_copy(src1.at[pl.ds(base, W)], sv[b])
        pltpu.sync_copy(dst1.at[pl.ds(base, W)], dv[b])

    def gath(b):
        return pltpu.async_copy(table.at[sv[b]], rows[b], sems[b])

    def wait(b):
        pltpu.make_async_copy(table.at[sv[b]], rows[b], sems[b]).wait()

    def scat(b):
        pltpu.sync_copy(rows[b], acc.at[dv[b]], add=True)

    load(0, 0)
    gath(0)

    def body(k, carry):
        t = 2 * k
        load(t + 1, 1)
        gath(1)
        wait(0)
        scat(0)
        load(t + 2, 0)
        gath(0)
        wait(1)
        scat(1)
        return carry
    lax.fori_loop(0, (sup - 1) // 2, body, 0)
    if sup % 2 == 1:
        wait(0)
        scat(0)
    else:
        load(sup - 1, 1)
        gath(1)
        wait(0)
        scat(0)
        wait(1)
        scat(1)


def _writeout(acc, out_hbm, s):
    n = N_NODE // NSUB
    pltpu.sync_copy(acc.at[pl.ds(s * n, n)], out_hbm.at[pl.ds(s * n, n)])


def _seg1_body(E):
    fn = _pick_accum(E)

    def body(t_lo, t_hi, src2, dst2, out_lo, out_hi,
             sv0, sv1, dv0, dv1, r0, r1, zbuf, acc, m0, m1, m2, m3):
        c = lax.axis_index("c")
        s = lax.axis_index("s")
        sv, dv, rows, sems = (sv0, sv1), (dv0, dv1), (r0, r1), (m0, m1)
        _zero_acc(acc, zbuf, r0, s, E)
        plsc.subcore_barrier()

        @pl.when(c == 0)
        def _():
            fn(t_lo, src2, dst2, sv, dv, rows, acc, sems, s, E)

        @pl.when(c == 1)
        def _():
            fn(t_hi, src2, dst2, sv, dv, rows, acc, sems, s, E)

        plsc.subcore_barrier()

        @pl.when(c == 0)
        def _():
            _writeout(acc, out_lo, s)

        @pl.when(c == 1)
        def _():
            _writeout(acc, out_hi, s)
    return body


def _seg3_body(E):
    fn = _pick_accum(E)

    def body(tu_lo, tu_hi, tv_lo, tv_hi, ty_lo, ty_hi,
             s0, s1, s2, d0, d1, d2, out_lo, out_hi,
             sv0, sv1, dv0, dv1, r0, r1, zbuf, acc, m0, m1, m2, m3):
        c = lax.axis_index("c")
        s = lax.axis_index("s")
        sv, dv, rows, sems = (sv0, sv1), (dv0, dv1), (r0, r1), (m0, m1)
        _zero_acc(acc, zbuf, r0, s, E)
        plsc.subcore_barrier()

        @pl.when(c == 0)
        def _():
            for t, sr, dr in ((tu_lo, s0, d0), (tv_lo, s1, d1), (ty_lo, s2, d2)):
                fn(t, sr, dr, sv, dv, rows, acc, sems, s, E)

        @pl.when(c == 1)
        def _():
            for t, sr, dr in ((tu_hi, s0, d0), (tv_hi, s1, d1), (ty_hi, s2, d2)):
                fn(t, sr, dr, sv, dv, rows, acc, sems, s, E)

        plsc.subcore_barrier()

        @pl.when(c == 0)
        def _():
            _writeout(acc, out_lo, s)

        @pl.when(c == 1)
        def _():
            _writeout(acc, out_hi, s)
    return body


def _is_static(E):
    K, S = _CHUNK[E]
    return (E // (K * S)) % NSUB == 0


def _pick_accum(E):
    return _accum_static if _is_static(E) else _accum


def _sc_scratch(E):
    K, S = _CHUNK[E]
    if _is_static(E):
        idx = pltpu.VMEM((S * K,), jnp.int32)
        rows = pltpu.VMEM((S * K, FH), jnp.float32)
        zbuf = pltpu.VMEM((8, FH), jnp.float32)
    else:
        idx = pltpu.VMEM((S, K), jnp.int32)
        rows = pltpu.VMEM((1, K, FH), jnp.float32)
        zbuf = pltpu.VMEM((_ZS, FH), jnp.float32)
    return [
        idx, idx, idx, idx, rows, rows, zbuf,
        pltpu.VMEM_SHARED((N_NODE, FH), jnp.float32),
        pltpu.SemaphoreType.DMA,
        pltpu.SemaphoreType.DMA,
        pltpu.SemaphoreType.DMA,
        pltpu.SemaphoreType.DMA,
    ]


def _mesh():
    return plsc.VectorSubcoreMesh(core_axis_name="c", subcore_axis_name="s",
                                  num_cores=2, num_subcores=NSUB)


_SC_PARAMS = pltpu.CompilerParams(use_tc_tiling_on_sc=False)

_HALF = jax.ShapeDtypeStruct((N_NODE, FH), jnp.float32)


@functools.cache
def _seg1(E):
    return pl.kernel(
        _seg1_body(E),
        out_type=(_HALF, _HALF),
        mesh=_mesh(),
        scratch_types=_sc_scratch(E),
        compiler_params=_SC_PARAMS,
    )


@functools.cache
def _seg3(E):
    return pl.kernel(
        _seg3_body(E),
        out_type=(_HALF, _HALF),
        mesh=_mesh(),
        scratch_types=_sc_scratch(E),
        compiler_params=_SC_PARAMS,
    )


_BR = 2000
_GRID = N_NODE // _BR


def _rows(d):
    return pl.BlockSpec((_BR, d), lambda i: (i, 0))


def _whole(shape):
    return pl.BlockSpec(shape, lambda i: tuple(0 for _ in shape))


def _split_store(o, lo_ref, hi_ref):
    lo_ref[...] = o[:, :FH]
    hi_ref[...] = o[:, FH:]


def _enc_body(x_ref, p_ref, mu_ref, sd_ref, we_ref, b_ref, wm_ref,
              lo_ref, hi_ref, mlo_ref, mhi_ref):
    xn = (x_ref[...] - mu_ref[...]) / sd_ref[...]
    f = jnp.concatenate([xn, p_ref[...]], axis=1)
    h = jnp.tanh(jnp.dot(f, we_ref[...], preferred_element_type=jnp.float32)
                 + b_ref[...])
    _split_store(h, lo_ref, hi_ref)
    _split_store(jnp.dot(h, wm_ref[...], preferred_element_type=jnp.float32),
                 mlo_ref, mhi_ref)


@jax.jit
def _enc(x, pos, mu, sd, we, b, wm):
    return pl.pallas_call(
        _enc_body,
        grid=(_GRID,),
        in_specs=[_rows(4), _rows(2), _whole((1, 4)), _whole((1, 4)),
                  _whole((6, HID)), _whole((1, HID)), _whole((HID, HID))],
        out_specs=[_rows(FH)] * 4,
        out_shape=[_HALF] * 4,
    )(x, pos, mu, sd, we, b, wm)


def _upd_body(hl_ref, hh_ref, al_ref, ah_ref, ws_ref, wn_ref,
              lo_ref, hi_ref, nlo_ref, nhi_ref):
    h = jnp.concatenate([hl_ref[...], hh_ref[...]], axis=1)
    a = jnp.concatenate([al_ref[...], ah_ref[...]], axis=1)
    nh = jnp.tanh(jnp.dot(h, ws_ref[...], preferred_element_type=jnp.float32) + a)
    _split_store(nh, lo_ref, hi_ref)
    _split_store(jnp.dot(nh, wn_ref[...], preferred_element_type=jnp.float32),
                 nlo_ref, nhi_ref)


@jax.jit
def _upd(hl, hh, al, ah, ws, wn):
    return pl.pallas_call(
        _upd_body,
        grid=(_GRID,),
        in_specs=[_rows(FH)] * 4 + [_whole((HID, HID))] * 2,
        out_specs=[_rows(FH)] * 4,
        out_shape=[_HALF] * 4,
    )(hl, hh, al, ah, ws, wn)


def _spupd_body(spz_ref, pl_ref, ph_ref, wb_ref,
                lo_ref, hi_ref, blo_ref, bhi_ref):
    p = jnp.concatenate([pl_ref[...], ph_ref[...]], axis=1)
    sp = jnp.tanh(p + spz_ref[0])
    _split_store(sp, lo_ref, hi_ref)
    _split_store(jnp.dot(sp, wb_ref[...], preferred_element_type=jnp.float32),
                 blo_ref, bhi_ref)


@jax.jit
def _spupd(spz, prel, preh, wb):
    return pl.pallas_call(
        _spupd_body,
        grid=(_GRID,),
        in_specs=[pl.BlockSpec(memory_space=pltpu.SMEM),
                  _rows(FH), _rows(FH), _whole((HID, HID))],
        out_specs=[_rows(FH)] * 4,
        out_shape=[_HALF] * 4,
    )(spz, prel, preh, wb)


def _back_body(hl_ref, hh_ref, bl_ref, bh_ref, wm_ref,
               lo_ref, hi_ref, mlo_ref, mhi_ref):
    h = jnp.concatenate([hl_ref[...], hh_ref[...]], axis=1)
    b = jnp.concatenate([bl_ref[...], bh_ref[...]], axis=1)
    nh = jnp.tanh(h + b)
    _split_store(nh, lo_ref, hi_ref)
    _split_store(jnp.dot(nh, wm_ref[...], preferred_element_type=jnp.float32),
                 mlo_ref, mhi_ref)


@jax.jit
def _backupd(hl, hh, bl, bh, wm):
    return pl.pallas_call(
        _back_body,
        grid=(_GRID,),
        in_specs=[_rows(FH)] * 4 + [_whole((HID, HID))],
        out_specs=[_rows(FH)] * 4,
        out_shape=[_HALF] * 4,
    )(hl, hh, bl, bh, wm)


def _heads_body(hl_ref, hh_ref, w6_ref, o_ref):
    h = jnp.concatenate([hl_ref[...], hh_ref[...]], axis=1)
    o_ref[...] = jnp.dot(h, w6_ref[...], preferred_element_type=jnp.float32)


@jax.jit
def _heads(hl, hh, w6):
    return pl.pallas_call(
        _heads_body,
        grid=(_GRID,),
        in_specs=[_rows(FH), _rows(FH), _whole((HID, 6))],
        out_specs=_rows(6),
        out_shape=jax.ShapeDtypeStruct((N_NODE, 6), jnp.float32),
    )(hl, hh, w6)


def _evt_body(sl_ref, sh_ref, we_ref, e_ref, v_ref):
    i = pl.program_id(0)

    @pl.when(i == 0)
    def _():
        e_ref[...] = jnp.zeros_like(e_ref)

    sp = jnp.concatenate([sl_ref[...], sh_ref[...]], axis=1)
    e_ref[...] += jnp.sum(sp, axis=0, keepdims=True)

    @pl.when(i == _GRID - 1)
    def _():
        e = e_ref[...] / np.float32(SP_NN)
        e_ref[...] = e
        v_ref[...] = jnp.dot(e, we_ref[...], preferred_element_type=jnp.float32)


@jax.jit
def _evt(sl, sh, we):
    return pl.pallas_call(
        _evt_body,
        grid=(_GRID,),
        in_specs=[_rows(FH), _rows(FH), _whole((HID, 5))],
        out_specs=[_whole((1, HID)), _whole((1, 5))],
        out_shape=[jax.ShapeDtypeStruct((1, HID), jnp.float32),
                   jax.ShapeDtypeStruct((1, 5), jnp.float32)],
    )(sl, sh, we)


def kernel(sp_num_nodes, u_x_dict, u_pos, v_x_dict, v_pos, y_x_dict, y_pos, evt_y,
           u_plane_u, u_nexus_sp, v_plane_v, v_nexus_sp, y_plane_y, y_nexus_sp,
           W_enc, b_enc, W_msg, W_self, W_nex, W_back, W_sem, W_filt, W_evt):
    f32, i32 = jnp.float32, jnp.int32
    planes = ('u', 'v', 'y')
    xs = {'u': u_x_dict, 'v': v_x_dict, 'y': y_x_dict}
    poss = {'u': u_pos, 'v': v_pos, 'y': y_pos}
    pe = {'u': u_plane_u.astype(i32), 'v': v_plane_v.astype(i32), 'y': y_plane_y.astype(i32)}
    ne = {'u': u_nexus_sp.astype(i32), 'v': v_nexus_sp.astype(i32), 'y': y_nexus_sp.astype(i32)}

    spz = (sp_num_nodes[0] - SP_NN).astype(f32).reshape(1)
    w2 = W_enc[4:6]
    b2 = b_enc.reshape(1, HID)

    mu = {p: jnp.asarray(_NORM[p][0]).reshape(1, 4) for p in planes}
    sd = {p: jnp.asarray(_NORM[p][1]).reshape(1, 4) for p in planes}
    hm = {}
    for p in planes:
        hm[p] = _enc(xs[p], poss[p], mu[p], sd[p], W_enc, b2, W_msg)

    Kn, Sn = _CHUNK[E_NX]
    ps = {p: pe[p][0] for p in planes}
    pd = {p: pe[p][1] for p in planes}
    ns = {p: ne[p][0].reshape(E_NX // Kn, Kn) for p in planes}
    nd = {p: ne[p][1].reshape(E_NX // Kn, Kn) for p in planes}

    for _ in range(3):
        agg = {p: _seg1(E_PL)(hm[p][2], hm[p][3], ps[p], pd[p]) for p in planes}
        hn = {p: _upd(hm[p][0], hm[p][1], agg[p][0], agg[p][1], W_self, W_nex)
              for p in planes}
        pre = _seg3(E_NX)(hn['u'][2], hn['u'][3], hn['v'][2], hn['v'][3],
                          hn['y'][2], hn['y'][3],
                          ns['u'], ns['v'], ns['y'],
                          nd['u'], nd['v'], nd['y'])
        sp = _spupd(spz, pre[0], pre[1], W_back)
        back = {p: _seg1(E_NX)(sp[2], sp[3], nd[p], ns[p]) for p in planes}
        hm = {p: _backupd(hn[p][0], hn[p][1], back[p][0], back[p][1], W_msg)
              for p in planes}

    w6 = jnp.concatenate([W_sem, W_filt], axis=1)
    x6 = {p: _heads(hm[p][0], hm[p][1], w6) for p in planes}
    e_evt, v_evt = _evt(sp[0], sp[1], W_evt)
    return (e_evt,
            x6['u'][:, :5], x6['v'][:, :5], x6['y'][:, :5],
            x6['u'][:, 5], x6['v'][:, 5], x6['y'][:, 5],
            v_evt)


# speedup vs baseline: 1.4217x; 1.0505x over previous
import functools

import jax
import jax.numpy as jnp
import numpy as np
from jax import lax
from jax.experimental import pallas as pl
from jax.experimental.pallas import tpu as pltpu
from jax.experimental.pallas import tpu_sc as plsc

N_NODE = 50000
SP_NN = 50000
E_PL = 800000
E_NX = 100000
HID = 64
FH = HID // 2
NSUB = 16

E_NXP = 102400
_CHUNK = {E_PL: (80, 5), E_NXP: (64, 5)}
_NTRASH = 8

_ZS = 625


def _norm_np():
    return {
        'u': np.array([[389.42752, 172.90794, 147.81108, 4.5563765], [147.1627, 78.01324, 228.31424, 2.2156637]], dtype=np.float32),
        'v': np.array([[368.83023, 173.01247, 154.14513, 4.449338], [145.29645, 80.54078, 282.34027, 1.8969047]], dtype=np.float32),
        'y': np.array([[546.2973, 172.77615, 116.974, 4.1647816], [283.47656, 73.99135, 115.49256, 1.4615369]], dtype=np.float32),
    }


_NORM = _norm_np()


def _zero_buf(buf, n):
    def row(i, carry):
        for j in range(FH // 16):
            buf[i, pl.ds(j * 16, 16)] = jnp.zeros((16,), jnp.float32)
        return carry
    lax.fori_loop(0, n, row, 0)


def _zero_acc(acc, zbuf, rows0, s, E):
    n = N_NODE // NSUB
    base = s * n
    if _is_static(E):
        K, S = _CHUNK[E]
        W = K * S
        _zero_buf(rows0, W)
        src, w = rows0, W
    else:
        _zero_buf(zbuf, _ZS)
        src, w = zbuf, _ZS
    full, rem = n // w, n % w
    for t in range(full):
        pltpu.sync_copy(src, acc.at[pl.ds(base + t * w, w)])
    if rem:
        pltpu.sync_copy(src.at[pl.ds(0, rem)],
                        acc.at[pl.ds(base + full * w, rem)])


def _accum(table, src2, dst2, sv, dv, rows, acc, sems, s, E):
    K, S = _CHUNK[E]
    nsuper = E // (K * S)
    cnt = (nsuper - s + NSUB - 1) // NSUB

    def body(k, carry):
        sg = s + k * NSUB
        pltpu.sync_copy(src2.at[pl.ds(sg * S, S)], sv[0])
        pltpu.sync_copy(dst2.at[pl.ds(sg * S, S)], dv[0])
        g = [None, None]
        g[0] = pltpu.async_copy(table.at[sv[0].at[0]], rows[0].at[0], sems[0])
        for j in range(S):
            b = j & 1
            if j + 1 < S:
                g[b ^ 1] = pltpu.async_copy(
                    table.at[sv[0].at[j + 1]], rows[b ^ 1].at[0], sems[b ^ 1])
            g[b].wait()
            pltpu.sync_copy(rows[b].at[0], acc.at[dv[0].at[j]], add=True)
        return carry
    lax.fori_loop(0, cnt, body, 0)


def _accum_static(table, src1, dst1, sv, dv, rows, acc, sems, s, E):
    K, S = _CHUNK[E]
    W = K * S
    sup = E // (W * NSUB)

    def load(t, b):
        base = pl.multiple_of((s + t * NSUB) * W, 8)
        pltpu.sync_copy(src1.at[pl.ds(base, W)], sv[b])
        pltpu.sync_copy(dst1.at[pl.ds(base, W)], dv[b])

    def gath(b):
        return pltpu.async_copy(table.at[sv[b]], rows[b], sems[b])

    def wait(b):
        pltpu.make_async_copy(table.at[sv[b]], rows[b], sems[b]).wait()

    def scat(b):
        pltpu.sync_copy(rows[b], acc.at[dv[b]], add=True)

    load(0, 0)
    gath(0)

    def body(k, carry):
        t = 2 * k
        load(t + 1, 1)
        gath(1)
        wait(0)
        scat(0)
        load(t + 2, 0)
        gath(0)
        wait(1)
        scat(1)
        return carry
    lax.fori_loop(0, (sup - 1) // 2, body, 0)
    if sup % 2 == 1:
        wait(0)
        scat(0)
    else:
        load(sup - 1, 1)
        gath(1)
        wait(0)
        scat(0)
        wait(1)
        scat(1)


def _writeout(acc, out_hbm, s):
    n = N_NODE // NSUB
    pltpu.sync_copy(acc.at[pl.ds(s * n, n)], out_hbm.at[pl.ds(s * n, n)])


def _seg1_body(E):
    fn = _pick_accum(E)

    def body(t_lo, t_hi, src2, dst2, out_lo, out_hi,
             sv0, sv1, dv0, dv1, r0, r1, zbuf, acc, m0, m1, m2, m3):
        c = lax.axis_index("c")
        s = lax.axis_index("s")
        sv, dv, rows, sems = (sv0, sv1), (dv0, dv1), (r0, r1), (m0, m1)
        _zero_acc(acc, zbuf, r0, s, E)
        plsc.subcore_barrier()

        @pl.when(c == 0)
        def _():
            fn(t_lo, src2, dst2, sv, dv, rows, acc, sems, s, E)

        @pl.when(c == 1)
        def _():
            fn(t_hi, src2, dst2, sv, dv, rows, acc, sems, s, E)

        plsc.subcore_barrier()

        @pl.when(c == 0)
        def _():
            _writeout(acc, out_lo, s)

        @pl.when(c == 1)
        def _():
            _writeout(acc, out_hi, s)
    return body


def _seg3_body(E):
    fn = _pick_accum(E)

    def body(tu_lo, tu_hi, tv_lo, tv_hi, ty_lo, ty_hi,
             s0, s1, s2, d0, d1, d2, out_lo, out_hi,
             sv0, sv1, dv0, dv1, r0, r1, zbuf, acc, m0, m1, m2, m3):
        c = lax.axis_index("c")
        s = lax.axis_index("s")
        sv, dv, rows, sems = (sv0, sv1), (dv0, dv1), (r0, r1), (m0, m1)
        _zero_acc(acc, zbuf, r0, s, E)
        plsc.subcore_barrier()

        @pl.when(c == 0)
        def _():
            for t, sr, dr in ((tu_lo, s0, d0), (tv_lo, s1, d1), (ty_lo, s2, d2)):
                fn(t, sr, dr, sv, dv, rows, acc, sems, s, E)

        @pl.when(c == 1)
        def _():
            for t, sr, dr in ((tu_hi, s0, d0), (tv_hi, s1, d1), (ty_hi, s2, d2)):
                fn(t, sr, dr, sv, dv, rows, acc, sems, s, E)

        plsc.subcore_barrier()

        @pl.when(c == 0)
        def _():
            _writeout(acc, out_lo, s)

        @pl.when(c == 1)
        def _():
            _writeout(acc, out_hi, s)
    return body


def _is_static(E):
    K, S = _CHUNK[E]
    return (E // (K * S)) % NSUB == 0


def _pick_accum(E):
    return _accum_static if _is_static(E) else _accum


def _sc_scratch(E):
    K, S = _CHUNK[E]
    if _is_static(E):
        idx = pltpu.VMEM((S * K,), jnp.int32)
        rows = pltpu.VMEM((S * K, FH), jnp.float32)
        zbuf = pltpu.VMEM((8, FH), jnp.float32)
    else:
        idx = pltpu.VMEM((S, K), jnp.int32)
        rows = pltpu.VMEM((1, K, FH), jnp.float32)
        zbuf = pltpu.VMEM((_ZS, FH), jnp.float32)
    return [
        idx, idx, idx, idx, rows, rows, zbuf,
        pltpu.VMEM_SHARED((N_NODE + _NTRASH, FH), jnp.float32),
        pltpu.SemaphoreType.DMA,
        pltpu.SemaphoreType.DMA,
        pltpu.SemaphoreType.DMA,
        pltpu.SemaphoreType.DMA,
    ]


def _mesh():
    return plsc.VectorSubcoreMesh(core_axis_name="c", subcore_axis_name="s",
                                  num_cores=2, num_subcores=NSUB)


_SC_PARAMS = pltpu.CompilerParams(use_tc_tiling_on_sc=False)

_HALF = jax.ShapeDtypeStruct((N_NODE, FH), jnp.float32)


@functools.cache
def _seg1(E):
    return pl.kernel(
        _seg1_body(E),
        out_type=(_HALF, _HALF),
        mesh=_mesh(),
        scratch_types=_sc_scratch(E),
        compiler_params=_SC_PARAMS,
    )


@functools.cache
def _seg3(E):
    return pl.kernel(
        _seg3_body(E),
        out_type=(_HALF, _HALF),
        mesh=_mesh(),
        scratch_types=_sc_scratch(E),
        compiler_params=_SC_PARAMS,
    )


_BR = 2000
_GRID = N_NODE // _BR


def _rows(d):
    return pl.BlockSpec((_BR, d), lambda i: (i, 0))


def _whole(shape):
    return pl.BlockSpec(shape, lambda i: tuple(0 for _ in shape))


def _split_store(o, lo_ref, hi_ref):
    lo_ref[...] = o[:, :FH]
    hi_ref[...] = o[:, FH:]


def _enc_body(x_ref, p_ref, mu_ref, sd_ref, we_ref, b_ref, wm_ref,
              lo_ref, hi_ref, mlo_ref, mhi_ref):
    xn = (x_ref[...] - mu_ref[...]) / sd_ref[...]
    f = jnp.concatenate([xn, p_ref[...]], axis=1)
    h = jnp.tanh(jnp.dot(f, we_ref[...], preferred_element_type=jnp.float32)
                 + b_ref[...])
    _split_store(h, lo_ref, hi_ref)
    _split_store(jnp.dot(h, wm_ref[...], preferred_element_type=jnp.float32),
                 mlo_ref, mhi_ref)


@jax.jit
def _enc(x, pos, mu, sd, we, b, wm):
    return pl.pallas_call(
        _enc_body,
        grid=(_GRID,),
        in_specs=[_rows(4), _rows(2), _whole((1, 4)), _whole((1, 4)),
                  _whole((6, HID)), _whole((1, HID)), _whole((HID, HID))],
        out_specs=[_rows(FH)] * 4,
        out_shape=[_HALF] * 4,
    )(x, pos, mu, sd, we, b, wm)


def _upd_body(hl_ref, hh_ref, al_ref, ah_ref, ws_ref, wn_ref,
              lo_ref, hi_ref, nlo_ref, nhi_ref):
    h = jnp.concatenate([hl_ref[...], hh_ref[...]], axis=1)
    a = jnp.concatenate([al_ref[...], ah_ref[...]], axis=1)
    nh = jnp.tanh(jnp.dot(h, ws_ref[...], preferred_element_type=jnp.float32) + a)
    _split_store(nh, lo_ref, hi_ref)
    _split_store(jnp.dot(nh, wn_ref[...], preferred_element_type=jnp.float32),
                 nlo_ref, nhi_ref)


@jax.jit
def _upd(hl, hh, al, ah, ws, wn):
    return pl.pallas_call(
        _upd_body,
        grid=(_GRID,),
        in_specs=[_rows(FH)] * 4 + [_whole((HID, HID))] * 2,
        out_specs=[_rows(FH)] * 4,
        out_shape=[_HALF] * 4,
    )(hl, hh, al, ah, ws, wn)


def _spupd_body(spz_ref, pl_ref, ph_ref, wb_ref,
                lo_ref, hi_ref, blo_ref, bhi_ref):
    p = jnp.concatenate([pl_ref[...], ph_ref[...]], axis=1)
    sp = jnp.tanh(p + spz_ref[0])
    _split_store(sp, lo_ref, hi_ref)
    _split_store(jnp.dot(sp, wb_ref[...], preferred_element_type=jnp.float32),
                 blo_ref, bhi_ref)


@jax.jit
def _spupd(spz, prel, preh, wb):
    return pl.pallas_call(
        _spupd_body,
        grid=(_GRID,),
        in_specs=[pl.BlockSpec(memory_space=pltpu.SMEM),
                  _rows(FH), _rows(FH), _whole((HID, HID))],
        out_specs=[_rows(FH)] * 4,
        out_shape=[_HALF] * 4,
    )(spz, prel, preh, wb)


def _back_body(hl_ref, hh_ref, bl_ref, bh_ref, wm_ref,
               lo_ref, hi_ref, mlo_ref, mhi_ref):
    h = jnp.concatenate([hl_ref[...], hh_ref[...]], axis=1)
    b = jnp.concatenate([bl_ref[...], bh_ref[...]], axis=1)
    nh = jnp.tanh(h + b)
    _split_store(nh, lo_ref, hi_ref)
    _split_store(jnp.dot(nh, wm_ref[...], preferred_element_type=jnp.float32),
                 mlo_ref, mhi_ref)


@jax.jit
def _backupd(hl, hh, bl, bh, wm):
    return pl.pallas_call(
        _back_body,
        grid=(_GRID,),
        in_specs=[_rows(FH)] * 4 + [_whole((HID, HID))],
        out_specs=[_rows(FH)] * 4,
        out_shape=[_HALF] * 4,
    )(hl, hh, bl, bh, wm)


def _heads_body(hl_ref, hh_ref, w6_ref, o_ref):
    h = jnp.concatenate([hl_ref[...], hh_ref[...]], axis=1)
    o_ref[...] = jnp.dot(h, w6_ref[...], preferred_element_type=jnp.float32)


@jax.jit
def _heads(hl, hh, w6):
    return pl.pallas_call(
        _heads_body,
        grid=(_GRID,),
        in_specs=[_rows(FH), _rows(FH), _whole((HID, 6))],
        out_specs=_rows(6),
        out_shape=jax.ShapeDtypeStruct((N_NODE, 6), jnp.float32),
    )(hl, hh, w6)


def _evt_body(sl_ref, sh_ref, we_ref, e_ref, v_ref):
    i = pl.program_id(0)

    @pl.when(i == 0)
    def _():
        e_ref[...] = jnp.zeros_like(e_ref)

    sp = jnp.concatenate([sl_ref[...], sh_ref[...]], axis=1)
    e_ref[...] += jnp.sum(sp, axis=0, keepdims=True)

    @pl.when(i == _GRID - 1)
    def _():
        e = e_ref[...] / np.float32(SP_NN)
        e_ref[...] = e
        v_ref[...] = jnp.dot(e, we_ref[...], preferred_element_type=jnp.float32)


@jax.jit
def _evt(sl, sh, we):
    return pl.pallas_call(
        _evt_body,
        grid=(_GRID,),
        in_specs=[_rows(FH), _rows(FH), _whole((HID, 5))],
        out_specs=[_whole((1, HID)), _whole((1, 5))],
        out_shape=[jax.ShapeDtypeStruct((1, HID), jnp.float32),
                   jax.ShapeDtypeStruct((1, 5), jnp.float32)],
    )(sl, sh, we)


def kernel(sp_num_nodes, u_x_dict, u_pos, v_x_dict, v_pos, y_x_dict, y_pos, evt_y,
           u_plane_u, u_nexus_sp, v_plane_v, v_nexus_sp, y_plane_y, y_nexus_sp,
           W_enc, b_enc, W_msg, W_self, W_nex, W_back, W_sem, W_filt, W_evt):
    f32, i32 = jnp.float32, jnp.int32
    planes = ('u', 'v', 'y')
    xs = {'u': u_x_dict, 'v': v_x_dict, 'y': y_x_dict}
    poss = {'u': u_pos, 'v': v_pos, 'y': y_pos}
    pe = {'u': u_plane_u.astype(i32), 'v': v_plane_v.astype(i32), 'y': y_plane_y.astype(i32)}
    ne = {'u': u_nexus_sp.astype(i32), 'v': v_nexus_sp.astype(i32), 'y': y_nexus_sp.astype(i32)}

    spz = (sp_num_nodes[0] - SP_NN).astype(f32).reshape(1)
    w2 = W_enc[4:6]
    b2 = b_enc.reshape(1, HID)

    mu = {p: jnp.asarray(_NORM[p][0]).reshape(1, 4) for p in planes}
    sd = {p: jnp.asarray(_NORM[p][1]).reshape(1, 4) for p in planes}
    hm = {}
    for p in planes:
        hm[p] = _enc(xs[p], poss[p], mu[p], sd[p], W_enc, b2, W_msg)

    npad = E_NXP - E_NX
    pad_src = jnp.asarray(np.arange(npad, dtype=np.int32) % 512)
    pad_dst = jnp.asarray(N_NODE + (np.arange(npad, dtype=np.int32) % _NTRASH))
    ps = {p: pe[p][0] for p in planes}
    pd = {p: pe[p][1] for p in planes}
    ns = {p: jnp.concatenate([ne[p][0], pad_src]) for p in planes}
    nd = {p: jnp.concatenate([ne[p][1], pad_dst]) for p in planes}
    bs = {p: jnp.concatenate([ne[p][1], pad_src]) for p in planes}
    bd = {p: jnp.concatenate([ne[p][0], pad_dst]) for p in planes}

    for _ in range(3):
        agg = {p: _seg1(E_PL)(hm[p][2], hm[p][3], ps[p], pd[p]) for p in planes}
        hn = {p: _upd(hm[p][0], hm[p][1], agg[p][0], agg[p][1], W_self, W_nex)
              for p in planes}
        pre = _seg3(E_NXP)(hn['u'][2], hn['u'][3], hn['v'][2], hn['v'][3],
                          hn['y'][2], hn['y'][3],
                          ns['u'], ns['v'], ns['y'],
                          nd['u'], nd['v'], nd['y'])
        sp = _spupd(spz, pre[0], pre[1], W_back)
        back = {p: _seg1(E_NXP)(sp[2], sp[3], bs[p], bd[p]) for p in planes}
        hm = {p: _backupd(hn[p][0], hn[p][1], back[p][0], back[p][1], W_msg)
              for p in planes}

    w6 = jnp.concatenate([W_sem, W_filt], axis=1)
    x6 = {p: _heads(hm[p][0], hm[p][1], w6) for p in planes}
    e_evt, v_evt = _evt(sp[0], sp[1], W_evt)
    return (e_evt,
            x6['u'][:, :5], x6['v'][:, :5], x6['y'][:, :5],
            x6['u'][:, 5], x6['v'][:, 5], x6['y'][:, 5],
            v_evt)
